# baseline (speedup 1.0000x reference)
import functools

import jax
import jax.numpy as jnp
from jax import lax
from jax.experimental import pallas as pl
from jax.experimental.pallas import tpu as pltpu
from jax.experimental.pallas import tpu_sc as plsc

DIM = 1024
HEADS = 4
NUM_KEYS = 256
TOPK = 16
DIM_HEAD = 256
HALF = DIM_HEAD // 2
T = 2048
TB = T // 2
NEG = -3.0e38


def _topk16(M, n):
    it = lax.broadcasted_iota(jnp.int32, M.shape, 0)
    ss, ii = [], []
    for k in range(TOPK):
        m = jnp.max(M, axis=0, keepdims=True)
        sel = jnp.min(jnp.where(M == m, it, n), axis=0, keepdims=True)
        ss.append(m)
        ii.append(sel)
        if k < TOPK - 1:
            M = jnp.where(it == sel, NEG, M)
    return jnp.concatenate(ss, axis=0), jnp.concatenate(ii, axis=0)


_CAND = [(i, j) for i in range(TOPK) for j in range(TOPK) if (i + 1) * (j + 1) <= TOPK]


def _stage12(qn_blk, keys_ref, idx_ref, attn_ref):
    stage1 = []
    for hp in range(2 * HEADS):
        h, p = hp // 2, hp % 2
        c0 = p * (HEADS * HALF) + h * HALF
        M = lax.dot_general(keys_ref[hp], qn_blk[:, c0:c0 + HALF],
                            (((1,), (1,)), ((), ())),
                            preferred_element_type=jnp.float32)
        stage1.append(_topk16(M, NUM_KEYS))
    for h in range(HEADS):
        s1, i1 = stage1[2 * h]
        s2, i2 = stage1[2 * h + 1]
        A = jnp.concatenate(
            [s1[i:i + 1, :] + s2[j:j + 1, :] for (i, j) in _CAND], axis=0)
        fs, fr = _topk16(A, len(_CAND))
        acc1 = jnp.zeros(fr.shape, jnp.int32)
        acc2 = jnp.zeros(fr.shape, jnp.int32)
        for r, (i, j) in enumerate(_CAND):
            acc1 = jnp.where(fr == r, i1[i:i + 1, :], acc1)
            acc2 = jnp.where(fr == r, i2[j:j + 1, :], acc2)
        vidx = acc1 * NUM_KEYS + acc2
        mx = jnp.max(fs, axis=0, keepdims=True)
        e = jnp.exp(fs - mx)
        at = e / jnp.sum(e, axis=0, keepdims=True)
        idx_ref[h * TOPK:(h + 1) * TOPK, :] = vidx
        attn_ref[h * TOPK:(h + 1) * TOPK, :] = at


def _tc_a_body(x_ref, wq_ref, keys_ref, g_ref, b_ref, idx_ref, attn_ref, qn_ref):
    q = jnp.dot(x_ref[...], wq_ref[...], preferred_element_type=jnp.float32)
    mean = jnp.mean(q, axis=0, keepdims=True)
    var = jnp.mean((q - mean) ** 2, axis=0, keepdims=True)
    qn = (q - mean) / jnp.sqrt(var + 1e-5) * g_ref[...] + b_ref[...]
    qn_ref[...] = qn
    _stage12(qn[:TB, :], keys_ref, idx_ref, attn_ref)


def _tc_b_body(qn_ref, keys_ref, idx_ref, attn_ref):
    _stage12(qn_ref[...], keys_ref, idx_ref, attn_ref)


def _tc_a(x2, W_q, keys8, g2, b2):
    return pl.pallas_call(
        _tc_a_body,
        out_shape=[
            jax.ShapeDtypeStruct((HEADS * TOPK, TB), jnp.int32),
            jax.ShapeDtypeStruct((HEADS * TOPK, TB), jnp.float32),
            jax.ShapeDtypeStruct((T, DIM), jnp.float32),
        ],
    )(x2, W_q, keys8, g2, b2)


def _tc_b(qn_hi, keys8):
    return pl.pallas_call(
        _tc_b_body,
        out_shape=[
            jax.ShapeDtypeStruct((HEADS * TOPK, TB), jnp.int32),
            jax.ShapeDtypeStruct((HEADS * TOPK, TB), jnp.float32),
        ],
    )(qn_hi, keys8)


NW = 32
KB = HEADS * TOPK
CH = DIM // 16
HB = KB // 2


def _sc_bag(values, idxT, attnT):
    tb = idxT.shape[0]
    tw = tb // NW
    mesh = plsc.VectorSubcoreMesh(core_axis_name="c", subcore_axis_name="s")

    @functools.partial(
        pl.kernel,
        mesh=mesh,
        out_type=jax.ShapeDtypeStruct((tb, DIM), jnp.float32),
        scratch_types=[
            pltpu.VMEM((tw * KB,), jnp.int32),
            pltpu.VMEM((tw * KB,), jnp.float32),
            pltpu.VMEM((HB, DIM), jnp.float32),
            pltpu.VMEM((HB, DIM), jnp.float32),
            pltpu.VMEM((DIM,), jnp.float32),
            pltpu.SemaphoreType.DMA,
            pltpu.SemaphoreType.DMA,
        ],
    )
    def bag(values_hbm, idx_hbm, attn_hbm, out_hbm, idx_v, attn_v, rows_a, rows_b,
            acc_v, sem_a, sem_b):
        wid = lax.axis_index("s") * 2 + lax.axis_index("c")
        base = wid * tw
        pltpu.sync_copy(idx_hbm.at[pl.ds(base * KB, tw * KB)], idx_v)
        pltpu.sync_copy(attn_hbm.at[pl.ds(base * KB, tw * KB)], attn_v)

        def gather(t, half, buf, sem):
            return pltpu.make_async_copy(
                values_hbm.at[idx_v.at[pl.ds(t * KB + half * HB, HB)]], buf, sem)

        gather(0, 0, rows_a, sem_a).start()
        gather(0, 1, rows_b, sem_b).start()

        def half_compute(t, half, buf):
            w0 = attn_v[pl.ds(t * KB + half * HB, 16)]
            w1 = attn_v[pl.ds(t * KB + half * HB + 16, 16)]

            def chunk(c, _):
                sl = pl.ds(c * 16, 16)
                if half == 0:
                    a = w0[0] * buf[0, sl]
                else:
                    a = acc_v[sl] + w0[0] * buf[0, sl]
                for kk in range(1, 16):
                    a = a + w0[kk] * buf[kk, sl]
                for kk in range(16):
                    a = a + w1[kk] * buf[16 + kk, sl]
                acc_v[sl] = a
                return 0

            lax.fori_loop(0, CH, chunk, 0, unroll=2)

        def token(t, carry):
            gather(t, 0, rows_a, sem_a).wait()
            half_compute(t, 0, rows_a)

            @pl.when(t + 1 < tw)
            def _():
                gather(t + 1, 0, rows_a, sem_a).start()

            gather(t, 1, rows_b, sem_b).wait()
            half_compute(t, 1, rows_b)

            @pl.when(t + 1 < tw)
            def _():
                gather(t + 1, 1, rows_b, sem_b).start()

            pltpu.sync_copy(acc_v, out_hbm.at[base + t])
            return carry

        lax.fori_loop(0, tw, token, 0)

    return bag(values, idxT.reshape(-1), attnT.reshape(-1))


def kernel(x, W_q, keys, values, bn_gamma, bn_beta):
    b, t, e = x.shape
    x2 = x.reshape(t, e)
    keys8 = keys.transpose(0, 2, 1, 3).reshape(2 * HEADS, NUM_KEYS, HALF)
    g2 = bn_gamma.reshape(1, DIM)
    b2 = bn_beta.reshape(1, DIM)
    idxA, attnA, qn = _tc_a(x2, W_q, keys8, g2, b2)
    outA = _sc_bag(values, idxA.T, attnA.T)
    idxB, attnB = _tc_b(qn[TB:, :], keys8)
    outB = _sc_bag(values, idxB.T, attnB.T)
    out = jnp.concatenate([outA, outB], axis=0)
    return out.reshape(b, t, e)


# speedup vs baseline: 1.0682x; 1.0682x over previous
import functools

import jax
import jax.numpy as jnp
from jax import lax
from jax.experimental import pallas as pl
from jax.experimental.pallas import tpu as pltpu
from jax.experimental.pallas import tpu_sc as plsc

DIM = 1024
HEADS = 4
NUM_KEYS = 256
TOPK = 16
DIM_HEAD = 256
HALF = DIM_HEAD // 2
T = 2048
NEG = -3.0e38


def _topk16(M, n):
    it = lax.broadcasted_iota(jnp.int32, M.shape, 0)
    ss, ii = [], []
    for k in range(TOPK):
        m = jnp.max(M, axis=0, keepdims=True)
        sel = jnp.min(jnp.where(M == m, it, n), axis=0, keepdims=True)
        ss.append(m)
        ii.append(sel)
        if k < TOPK - 1:
            M = jnp.where(it == sel, NEG, M)
    return jnp.concatenate(ss, axis=0), jnp.concatenate(ii, axis=0)


_CAND = [(i, j) for i in range(TOPK) for j in range(TOPK) if (i + 1) * (j + 1) <= TOPK]


def _tc_body(x_ref, wq_ref, keys_ref, g_ref, b_ref, idx_ref, attn_ref):
    xv = x_ref[...]
    q = jnp.dot(xv, wq_ref[...], preferred_element_type=jnp.float32)
    mean = jnp.mean(q, axis=0, keepdims=True)
    var = jnp.mean((q - mean) ** 2, axis=0, keepdims=True)
    qn = (q - mean) / jnp.sqrt(var + 1e-5) * g_ref[...] + b_ref[...]

    stage1 = []
    for hp in range(2 * HEADS):
        h, p = hp // 2, hp % 2
        c0 = p * (HEADS * HALF) + h * HALF
        qcols = qn[:, c0:c0 + HALF]
        K = keys_ref[hp]
        M = lax.dot_general(K, qcols, (((1,), (1,)), ((), ())),
                            preferred_element_type=jnp.float32)
        stage1.append(_topk16(M, NUM_KEYS))

    for h in range(HEADS):
        s1, i1 = stage1[2 * h]
        s2, i2 = stage1[2 * h + 1]
        A = jnp.concatenate(
            [s1[i:i + 1, :] + s2[j:j + 1, :] for (i, j) in _CAND], axis=0)
        fs, fr = _topk16(A, len(_CAND))
        acc1 = jnp.zeros(fr.shape, jnp.int32)
        acc2 = jnp.zeros(fr.shape, jnp.int32)
        for r, (i, j) in enumerate(_CAND):
            acc1 = jnp.where(fr == r, i1[i:i + 1, :], acc1)
            acc2 = jnp.where(fr == r, i2[j:j + 1, :], acc2)
        vidx = acc1 * NUM_KEYS + acc2
        mx = jnp.max(fs, axis=0, keepdims=True)
        e = jnp.exp(fs - mx)
        at = e / jnp.sum(e, axis=0, keepdims=True)
        idx_ref[h * TOPK:(h + 1) * TOPK, :] = vidx
        attn_ref[h * TOPK:(h + 1) * TOPK, :] = at


def _tc_call(x2, W_q, keys8, g2, b2):
    return pl.pallas_call(
        _tc_body,
        out_shape=[
            jax.ShapeDtypeStruct((HEADS * TOPK, T), jnp.int32),
            jax.ShapeDtypeStruct((HEADS * TOPK, T), jnp.float32),
        ],
    )(x2, W_q, keys8, g2, b2)


NW = 32
TW = T // NW
KB = HEADS * TOPK
CH = DIM // 16


HB = KB // 2


def _sc_bag(values, idxT, attnT):
    mesh = plsc.VectorSubcoreMesh(core_axis_name="c", subcore_axis_name="s")

    @functools.partial(
        pl.kernel,
        mesh=mesh,
        out_type=jax.ShapeDtypeStruct((T, DIM), jnp.float32),
        scratch_types=[
            pltpu.VMEM((TW * KB,), jnp.int32),
            pltpu.VMEM((TW * KB,), jnp.float32),
            pltpu.VMEM((HB, DIM), jnp.float32),
            pltpu.VMEM((HB, DIM), jnp.float32),
            pltpu.VMEM((DIM,), jnp.float32),
            pltpu.SemaphoreType.DMA,
            pltpu.SemaphoreType.DMA,
        ],
    )
    def bag(values_hbm, idx_hbm, attn_hbm, out_hbm, idx_v, attn_v, rows_a, rows_b,
            acc_v, sem_a, sem_b):
        wid = lax.axis_index("s") * 2 + lax.axis_index("c")
        base = wid * TW
        pltpu.sync_copy(idx_hbm.at[pl.ds(base * KB, TW * KB)], idx_v)
        pltpu.sync_copy(attn_hbm.at[pl.ds(base * KB, TW * KB)], attn_v)

        def gather(t, half, buf, sem):
            return pltpu.make_async_copy(
                values_hbm.at[idx_v.at[pl.ds(t * KB + half * HB, HB)]], buf, sem)

        gather(0, 0, rows_a, sem_a).start()
        gather(0, 1, rows_b, sem_b).start()

        def half_compute(t, half, buf):
            w0 = attn_v[pl.ds(t * KB + half * HB, 16)]
            w1 = attn_v[pl.ds(t * KB + half * HB + 16, 16)]

            def chunk(c, _):
                sl = pl.ds(c * 16, 16)
                if half == 0:
                    a = w0[0] * buf[0, sl]
                else:
                    a = acc_v[sl] + w0[0] * buf[0, sl]
                for kk in range(1, 16):
                    a = a + w0[kk] * buf[kk, sl]
                for kk in range(16):
                    a = a + w1[kk] * buf[16 + kk, sl]
                acc_v[sl] = a
                return 0

            lax.fori_loop(0, CH, chunk, 0, unroll=2)

        def token(t, carry):
            gather(t, 0, rows_a, sem_a).wait()
            half_compute(t, 0, rows_a)

            @pl.when(t + 1 < TW)
            def _():
                gather(t + 1, 0, rows_a, sem_a).start()

            gather(t, 1, rows_b, sem_b).wait()
            half_compute(t, 1, rows_b)

            @pl.when(t + 1 < TW)
            def _():
                gather(t + 1, 1, rows_b, sem_b).start()

            pltpu.sync_copy(acc_v, out_hbm.at[base + t])
            return carry

        lax.fori_loop(0, TW, token, 0)

    return bag(values, idxT.reshape(-1), attnT.reshape(-1))


def kernel(x, W_q, keys, values, bn_gamma, bn_beta):
    b, t, e = x.shape
    x2 = x.reshape(t, e)
    keys8 = keys.transpose(0, 2, 1, 3).reshape(2 * HEADS, NUM_KEYS, HALF)
    g2 = bn_gamma.reshape(1, DIM)
    b2 = bn_beta.reshape(1, DIM)
    idx64, attn64 = _tc_call(x2, W_q, keys8, g2, b2)
    idxT = idx64.T
    attnT = attn64.T
    out = _sc_bag(values, idxT, attnT)
    return out.reshape(b, t, e)
